# baseline (speedup 1.0000x reference)
import jax
import jax.numpy as jnp
from jax import lax
from jax.experimental import pallas as pl
from jax.experimental.pallas import tpu as pltpu

VOCAB = 100000
EMB = 64
CTX = 200
FLAT = CTX * EMB

_KBLK = 1280
_KSTEPS = FLAT // _KBLK
_ROWS_PER_K = _KBLK // EMB
_GRID = _KSTEPS + 1


def _body(idx_ref, embT_ref, w3t_hbm, waa_ref, wab_ref, ba_ref, w1_ref, b1_ref,
          w2t_ref, b2_ref, b3_ref, out_ref, bcols_ref, xcol_ref, acc_ref,
          h2_ref, w3t_vmem, sems, w3sem):
    j = pl.program_id(0)

    @pl.when(j == 0)
    def _():
        acc_ref[...] = jnp.zeros_like(acc_ref)
        for t in range(CTX):
            base = pl.multiple_of((idx_ref[t] // 128) * 128, 128)
            pltpu.make_async_copy(
                embT_ref.at[:, pl.ds(base, 128)],
                bcols_ref.at[:, pl.ds(t * 128, 128)],
                sems.at[t // _ROWS_PER_K]).start()
        pltpu.make_async_copy(w3t_hbm, w3t_vmem, w3sem).start()

    @pl.when(j < _KSTEPS)
    def _():
        for _ in range(_ROWS_PER_K):
            pltpu.make_async_copy(
                embT_ref.at[:, pl.ds(0, 128)],
                bcols_ref.at[:, pl.ds(0, 128)],
                sems.at[j]).wait()

        lane_iota = lax.broadcasted_iota(jnp.int32, (1, 128), 1)

        def extract(i, carry):
            t = j * _ROWS_PER_K + i
            lane = idx_ref[t] % 128
            off = pl.multiple_of(t * 128, 128)
            blk = bcols_ref[:, pl.ds(off, 128)]
            oh = (lane_iota == lane).astype(jnp.float32)
            v = jnp.sum(blk * oh, axis=1, keepdims=True)
            xcol_ref[pl.ds(t * EMB, EMB), :] = v
            return carry

        lax.fori_loop(0, _ROWS_PER_K, extract, 0)

        xs = xcol_ref[pl.ds(j * _KBLK, _KBLK), :]
        acc_ref[pl.ds(0, 256), :] += lax.dot_general(
            waa_ref[...], xs,
            (((1,), (0,)), ((), ())), preferred_element_type=jnp.float32)
        acc_ref[pl.ds(256, 256), :] += lax.dot_general(
            wab_ref[...], xs,
            (((1,), (0,)), ((), ())), preferred_element_type=jnp.float32)

    @pl.when(j == _KSTEPS - 1)
    def _():
        h0 = acc_ref[...] + ba_ref[...]
        h1 = jax.nn.relu(
            lax.dot_general(w1_ref[...], h0, (((1,), (0,)), ((), ())),
                            preferred_element_type=jnp.float32) + b1_ref[...])
        h2_ref[...] = jax.nn.relu(
            lax.dot_general(h1, w2t_ref[...], (((0,), (0,)), ((), ())),
                            preferred_element_type=jnp.float32) + b2_ref[...])

    @pl.when(j == _KSTEPS)
    def _():
        pltpu.make_async_copy(w3t_hbm, w3t_vmem, w3sem).wait()
        logits = lax.dot_general(
            h2_ref[...], w3t_vmem[...], (((1,), (0,)), ((), ())),
            preferred_element_type=jnp.float32) + b3_ref[...]
        m = jnp.max(logits)
        lse = m + jnp.log(jnp.sum(jnp.exp(logits - m)))
        out_ref[...] = logits - lse


def kernel(inputs, emb, W_aug, b_aug, W1, b1, W2, b2, W3, b3):
    idx = inputs.astype(jnp.int32)
    return pl.pallas_call(
        _body,
        grid=(_GRID,),
        in_specs=[
            pl.BlockSpec(memory_space=pltpu.SMEM),
            pl.BlockSpec(memory_space=pltpu.MemorySpace.HBM),
            pl.BlockSpec(memory_space=pltpu.MemorySpace.HBM),
            pl.BlockSpec((256, _KBLK),
                         lambda j: (0, jnp.minimum(j, _KSTEPS - 1))),
            pl.BlockSpec((256, _KBLK),
                         lambda j: (1, jnp.minimum(j, _KSTEPS - 1))),
            pl.BlockSpec((512, 1), lambda j: (0, 0)),
            pl.BlockSpec((128, 512), lambda j: (0, 0)),
            pl.BlockSpec((128, 1), lambda j: (0, 0)),
            pl.BlockSpec((128, 64), lambda j: (0, 0)),
            pl.BlockSpec((1, 64), lambda j: (0, 0)),
            pl.BlockSpec((1, VOCAB), lambda j: (0, 0)),
        ],
        out_specs=pl.BlockSpec((1, VOCAB), lambda j: (0, 0)),
        out_shape=jax.ShapeDtypeStruct((1, VOCAB), jnp.float32),
        scratch_shapes=[
            pltpu.VMEM((EMB, CTX * 128), jnp.float32),
            pltpu.VMEM((FLAT, 1), jnp.float32),
            pltpu.VMEM((512, 1), jnp.float32),
            pltpu.VMEM((1, EMB), jnp.float32),
            pltpu.VMEM((EMB, VOCAB), jnp.float32),
            pltpu.SemaphoreType.DMA((_KSTEPS,)),
            pltpu.SemaphoreType.DMA,
        ],
        compiler_params=pltpu.CompilerParams(
            vmem_limit_bytes=100 * 1024 * 1024),
    )(idx, emb.T, W3.T, W_aug, W_aug, b_aug.reshape(512, 1), W1,
      b1.reshape(128, 1), W2.T, b2.reshape(1, 64), b3.reshape(1, VOCAB))
